# baseline (speedup 1.0000x reference)
import functools

import jax
import jax.numpy as jnp
from jax import lax
from jax.experimental import pallas as pl
from jax.experimental.pallas import tpu as pltpu
from jax.experimental.pallas import tpu_sc as plsc

B, L, H = 1024, 200, 128
NW = 32
TOK = B * L
TPW = TOK // NW
CHUNK = 128
NCHUNK = TPW // CHUNK

_MESH = plsc.VectorSubcoreMesh(core_axis_name="c", subcore_axis_name="s")

_F32 = jnp.float32
_I32 = jnp.int32

_BCAST_DNUMS = lax.GatherDimensionNumbers(
    offset_dims=(), collapsed_slice_dims=(0,), start_index_map=(0,))


def _lane_bcast(vec, k):
    idx = jnp.full((16, 1), k, _I32)
    return lax.gather(vec, idx, _BCAST_DNUMS, slice_sizes=(1,),
                      mode=lax.GatherScatterMode.PROMISE_IN_BOUNDS)


@functools.partial(
    pl.kernel,
    out_type=jax.ShapeDtypeStruct((TOK, H), _F32),
    mesh=_MESH,
    scratch_types=[
        pltpu.VMEM((TPW + 2 * CHUNK,), _I32),
        pltpu.VMEM((TPW,), _I32),
        pltpu.VMEM((L, H), _F32),
        pltpu.VMEM((H,), _F32),
        [pltpu.VMEM((CHUNK, H), _F32)] * 2,
        [pltpu.VMEM((CHUNK, H), _F32)] * 2,
        [pltpu.SemaphoreType.DMA] * 2,
        [pltpu.SemaphoreType.DMA] * 2,
    ],
)
def _sc_embed(ids_hbm, tt_hbm, w_hbm, p0_hbm, d_hbm, out_hbm,
              idw, ttb, p0res, dbuf, wbufs, obufs, sem_w, sem_o):
    c = lax.axis_index("c")
    s = lax.axis_index("s")
    wid = s * 2 + c
    base0 = wid * TPW
    sls = [pl.ds(j * 16, 16) for j in range(H // 16)]

    pltpu.sync_copy(ids_hbm.at[pl.ds(base0, TPW)], idw.at[pl.ds(0, TPW)])
    for i in range(2 * CHUNK // 16):
        idw[pl.ds(TPW + i * 16, 16)] = jnp.zeros((16,), _I32)
    pltpu.sync_copy(tt_hbm.at[pl.ds(base0, TPW)], ttb)
    pltpu.sync_copy(p0_hbm, p0res)
    pltpu.sync_copy(d_hbm, dbuf)
    dvals = [dbuf[sl] for sl in sls]

    def launch(g, b):
        pltpu.async_copy(w_hbm.at[idw.at[pl.ds(g * CHUNK, CHUNK)]],
                         wbufs[b], sem_w[b])

    def wait_gather(b):
        pltpu.make_async_copy(w_hbm.at[pl.ds(0, CHUNK)], wbufs[b],
                              sem_w[b]).wait()

    def wait_out(b):
        pltpu.make_async_copy(obufs[b], out_hbm.at[pl.ds(0, CHUNK)],
                              sem_o[b]).wait()

    def compute(g, b):
        wb, ob = wbufs[b], obufs[b]
        off = lax.rem(g * CHUNK, L)

        @plsc.parallel_loop(0, CHUNK, unroll=2)
        def row_body(r):
            pos = off + r
            pos = lax.select(pos >= L, pos - L, pos)
            rbase = (r // 16) * 16
            ttv = ttb[pl.ds(g * CHUNK + rbase, 16)]
            tk = _lane_bcast(ttv, r - rbase).astype(_F32)
            ws = [wb[r, sl] for sl in sls]
            ps = [p0res[pos, sl] for sl in sls]
            for j in range(H // 16):
                v = ws[j] + ps[j] + tk * dvals[j]
                ob[r, sls[j]] = jnp.minimum(jnp.maximum(v, -1.0), 1.0)

    def store(g, b):
        pltpu.async_copy(obufs[b], out_hbm.at[pl.ds(base0 + g * CHUNK, CHUNK)],
                         sem_o[b])

    def step(g, b, *, first):
        wait_gather(b)
        if not first:
            wait_out(b)
        compute(g, b)
        store(g, b)
        launch(g + 2, b)

    launch(0, 0)
    launch(1, 1)
    step(0, 0, first=True)
    step(1, 1, first=True)

    def pair_body(go, carry):
        step(2 * go, 0, first=False)
        step(2 * go + 1, 1, first=False)
        return carry

    lax.fori_loop(1, NCHUNK // 2, pair_body, 0)

    wait_gather(0)
    wait_gather(1)
    wait_out(0)
    wait_out(1)


def kernel(input_ids, attention_mask, token_type_ids, word_embeddings,
           position_embeddings, token_type_embeddings):
    del attention_mask
    ids = input_ids.reshape(TOK).astype(_I32)
    tt = token_type_ids.reshape(TOK).astype(_I32)
    p0 = position_embeddings[:L] + token_type_embeddings[0]
    d = token_type_embeddings[1] - token_type_embeddings[0]
    out = _sc_embed(ids, tt, word_embeddings, p0, d)
    return out.reshape(B, L, H)


# speedup vs baseline: 3.7774x; 3.7774x over previous
import functools

import jax
import jax.numpy as jnp
from jax import lax
from jax.experimental import pallas as pl
from jax.experimental.pallas import tpu as pltpu
from jax.experimental.pallas import tpu_sc as plsc

B, L, H = 1024, 200, 128
NW = 32
TOK = B * L
TPW = TOK // NW
CHUNK = 64
NCHUNK = TPW // CHUNK
NB = 4

_MESH = plsc.VectorSubcoreMesh(core_axis_name="c", subcore_axis_name="s")

_F32 = jnp.float32
_I32 = jnp.int32

_BCAST_DNUMS = lax.GatherDimensionNumbers(
    offset_dims=(), collapsed_slice_dims=(0,), start_index_map=(0,))


def _lane_bcast(vec, k):
    idx = jnp.full((16, 1), k, _I32)
    return lax.gather(vec, idx, _BCAST_DNUMS, slice_sizes=(1,),
                      mode=lax.GatherScatterMode.PROMISE_IN_BOUNDS)


@functools.partial(
    pl.kernel,
    out_type=jax.ShapeDtypeStruct((TOK, H), _F32),
    mesh=_MESH,
    scratch_types=[
        pltpu.VMEM((TPW + NB * CHUNK,), _I32),
        pltpu.VMEM((TPW,), _I32),
        pltpu.VMEM((L, H), _F32),
        pltpu.VMEM((H,), _F32),
        [pltpu.VMEM((CHUNK, H), _F32)] * NB,
        [pltpu.VMEM((CHUNK, H), _F32)] * NB,
        [pltpu.SemaphoreType.DMA] * NB,
        [pltpu.SemaphoreType.DMA] * NB,
    ],
)
def _sc_embed(ids_hbm, tt_hbm, w_hbm, p0_hbm, d_hbm, out_hbm,
              idw, ttb, p0res, dbuf, wbufs, obufs, sem_w, sem_o):
    c = lax.axis_index("c")
    s = lax.axis_index("s")
    wid = s * 2 + c
    base0 = wid * TPW
    sls = [pl.ds(j * 16, 16) for j in range(H // 16)]
    iota = lax.iota(_I32, 16)

    pltpu.sync_copy(ids_hbm.at[pl.ds(base0, TPW)], idw.at[pl.ds(0, TPW)])
    for i in range(NB * CHUNK // 16):
        idw[pl.ds(TPW + i * 16, 16)] = iota + (i * 16)
    pltpu.sync_copy(tt_hbm.at[pl.ds(base0, TPW)], ttb)
    pltpu.sync_copy(p0_hbm, p0res)
    pltpu.sync_copy(d_hbm, dbuf)
    dvals = [dbuf[sl] for sl in sls]

    def launch(g, b):
        pltpu.async_copy(w_hbm.at[idw.at[pl.ds(g * CHUNK, CHUNK)]],
                         wbufs[b], sem_w[b])

    def wait_gather(b):
        pltpu.make_async_copy(w_hbm.at[pl.ds(0, CHUNK)], wbufs[b],
                              sem_w[b]).wait()

    def wait_out(b):
        pltpu.make_async_copy(obufs[b], out_hbm.at[pl.ds(0, CHUNK)],
                              sem_o[b]).wait()

    def compute(g, b):
        wb, ob = wbufs[b], obufs[b]
        off = lax.rem(g * CHUNK, L)

        @plsc.parallel_loop(0, CHUNK, unroll=2)
        def row_body(r):
            pos = off + r
            pos = lax.select(pos >= L, pos - L, pos)
            rbase = (r // 16) * 16
            ttv = ttb[pl.ds(g * CHUNK + rbase, 16)]
            tk = _lane_bcast(ttv, r - rbase).astype(_F32)
            ws = [wb[r, sl] for sl in sls]
            ps = [p0res[pos, sl] for sl in sls]
            for j in range(H // 16):
                v = ws[j] + ps[j] + tk * dvals[j]
                ob[r, sls[j]] = jnp.minimum(jnp.maximum(v, -1.0), 1.0)

    def store(g, b):
        pltpu.async_copy(obufs[b], out_hbm.at[pl.ds(base0 + g * CHUNK, CHUNK)],
                         sem_o[b])

    def step(g, b, *, first):
        wait_gather(b)
        if not first:
            wait_out(b)
        compute(g, b)
        store(g, b)
        launch(g + NB, b)

    for b in range(NB):
        launch(b, b)
    for b in range(NB):
        step(b, b, first=True)

    def quad_body(go, carry):
        for b in range(NB):
            step(NB * go + b, b, first=False)
        return carry

    lax.fori_loop(1, NCHUNK // NB, quad_body, 0)

    for b in range(NB):
        wait_gather(b)
        wait_out(b)


def kernel(input_ids, attention_mask, token_type_ids, word_embeddings,
           position_embeddings, token_type_embeddings):
    del attention_mask
    ids = input_ids.reshape(TOK).astype(_I32)
    tt = token_type_ids.reshape(TOK).astype(_I32)
    p0 = position_embeddings[:L] + token_type_embeddings[0]
    d = token_type_embeddings[1] - token_type_embeddings[0]
    out = _sc_embed(ids, tt, word_embeddings, p0, d)
    return out.reshape(B, L, H)


# speedup vs baseline: 3.7852x; 1.0021x over previous
import functools

import jax
import jax.numpy as jnp
from jax import lax
from jax.experimental import pallas as pl
from jax.experimental.pallas import tpu as pltpu
from jax.experimental.pallas import tpu_sc as plsc

B, L, H = 1024, 200, 128
NW = 32
TOK = B * L
TPW = TOK // NW
CHUNK = 64
NCHUNK = TPW // CHUNK
NB = 5

_MESH = plsc.VectorSubcoreMesh(core_axis_name="c", subcore_axis_name="s")

_F32 = jnp.float32
_I32 = jnp.int32

_BCAST_DNUMS = lax.GatherDimensionNumbers(
    offset_dims=(), collapsed_slice_dims=(0,), start_index_map=(0,))


def _lane_bcast(vec, k):
    idx = jnp.full((16, 1), k, _I32)
    return lax.gather(vec, idx, _BCAST_DNUMS, slice_sizes=(1,),
                      mode=lax.GatherScatterMode.PROMISE_IN_BOUNDS)


@functools.partial(
    pl.kernel,
    out_type=jax.ShapeDtypeStruct((TOK, H), _F32),
    mesh=_MESH,
    scratch_types=[
        pltpu.VMEM((TPW + NB * CHUNK,), _I32),
        pltpu.VMEM((TPW,), _I32),
        pltpu.VMEM((L, H), _F32),
        pltpu.VMEM((H,), _F32),
        [pltpu.VMEM((CHUNK, H), _F32)] * NB,
        [pltpu.VMEM((CHUNK, H), _F32)] * NB,
        [pltpu.SemaphoreType.DMA] * NB,
        [pltpu.SemaphoreType.DMA] * NB,
    ],
)
def _sc_embed(ids_hbm, tt_hbm, w_hbm, p0_hbm, d_hbm, out_hbm,
              idw, ttb, p0res, dbuf, wbufs, obufs, sem_w, sem_o):
    c = lax.axis_index("c")
    s = lax.axis_index("s")
    wid = s * 2 + c
    base0 = wid * TPW
    sls = [pl.ds(j * 16, 16) for j in range(H // 16)]
    iota = lax.iota(_I32, 16)

    pltpu.sync_copy(ids_hbm.at[pl.ds(base0, TPW)], idw.at[pl.ds(0, TPW)])
    for i in range(NB * CHUNK // 16):
        idw[pl.ds(TPW + i * 16, 16)] = iota + (i * 16)
    pltpu.sync_copy(tt_hbm.at[pl.ds(base0, TPW)], ttb)
    pltpu.sync_copy(p0_hbm, p0res)
    pltpu.sync_copy(d_hbm, dbuf)
    dvals = [dbuf[sl] for sl in sls]

    def launch(g, b):
        pltpu.async_copy(w_hbm.at[idw.at[pl.ds(g * CHUNK, CHUNK)]],
                         wbufs[b], sem_w[b])

    def wait_gather(b):
        pltpu.make_async_copy(w_hbm.at[pl.ds(0, CHUNK)], wbufs[b],
                              sem_w[b]).wait()

    def wait_out(b):
        pltpu.make_async_copy(obufs[b], out_hbm.at[pl.ds(0, CHUNK)],
                              sem_o[b]).wait()

    def compute(g, b):
        wb, ob = wbufs[b], obufs[b]
        off = lax.rem(g * CHUNK, L)

        @plsc.parallel_loop(0, CHUNK, unroll=2)
        def row_body(r):
            pos = off + r
            pos = lax.select(pos >= L, pos - L, pos)
            rbase = (r // 16) * 16
            ttv = ttb[pl.ds(g * CHUNK + rbase, 16)]
            tk = _lane_bcast(ttv, r - rbase).astype(_F32)
            ws = [wb[r, sl] for sl in sls]
            ps = [p0res[pos, sl] for sl in sls]
            for j in range(H // 16):
                v = ws[j] + ps[j] + tk * dvals[j]
                ob[r, sls[j]] = jnp.minimum(jnp.maximum(v, -1.0), 1.0)

    def store(g, b):
        pltpu.async_copy(obufs[b], out_hbm.at[pl.ds(base0 + g * CHUNK, CHUNK)],
                         sem_o[b])

    def step(g, b, *, first):
        wait_gather(b)
        if not first:
            wait_out(b)
        compute(g, b)
        store(g, b)
        launch(g + NB, b)

    for b in range(NB):
        launch(b, b)
    for b in range(NB):
        step(b, b, first=True)

    def quad_body(go, carry):
        for b in range(NB):
            step(NB * go + b, b, first=False)
        return carry

    lax.fori_loop(1, NCHUNK // NB, quad_body, 0)

    for b in range(NB):
        wait_gather(b)
        wait_out(b)


def kernel(input_ids, attention_mask, token_type_ids, word_embeddings,
           position_embeddings, token_type_embeddings):
    del attention_mask
    ids = input_ids.reshape(TOK).astype(_I32)
    tt = token_type_ids.reshape(TOK).astype(_I32)
    p0 = position_embeddings[:L] + token_type_embeddings[0]
    d = token_type_embeddings[1] - token_type_embeddings[0]
    out = _sc_embed(ids, tt, word_embeddings, p0, d)
    return out.reshape(B, L, H)


# speedup vs baseline: 3.9461x; 1.0425x over previous
import functools

import jax
import jax.numpy as jnp
from jax import lax
from jax.experimental import pallas as pl
from jax.experimental.pallas import tpu as pltpu
from jax.experimental.pallas import tpu_sc as plsc

B, L, H = 1024, 200, 128
NW = 32
TOK = B * L
TPW = TOK // NW
CHUNK = 64
NCHUNK = TPW // CHUNK
NB = 5

_MESH = plsc.VectorSubcoreMesh(core_axis_name="c", subcore_axis_name="s")

_F32 = jnp.float32
_I32 = jnp.int32

_BCAST_DNUMS = lax.GatherDimensionNumbers(
    offset_dims=(), collapsed_slice_dims=(0,), start_index_map=(0,))


def _lane_bcast(vec, k):
    idx = jnp.full((16, 1), k, _I32)
    return lax.gather(vec, idx, _BCAST_DNUMS, slice_sizes=(1,),
                      mode=lax.GatherScatterMode.PROMISE_IN_BOUNDS)


@functools.partial(
    pl.kernel,
    out_type=jax.ShapeDtypeStruct((TOK, H), _F32),
    mesh=_MESH,
    scratch_types=[
        pltpu.VMEM((TPW,), _I32),
        pltpu.VMEM((TPW,), _I32),
        pltpu.VMEM((L, H), _F32),
        pltpu.VMEM((H,), _F32),
        [pltpu.VMEM((CHUNK, H), _F32)] * NB,
        [pltpu.VMEM((CHUNK, H), _F32)] * NB,
        [pltpu.SemaphoreType.DMA] * NB,
        [pltpu.SemaphoreType.DMA] * NB,
    ],
)
def _sc_embed(ids_hbm, tt_hbm, w_hbm, p0_hbm, d_hbm, out_hbm,
              idw, ttb, p0res, dbuf, wbufs, obufs, sem_w, sem_o):
    c = lax.axis_index("c")
    s = lax.axis_index("s")
    wid = s * 2 + c
    base0 = wid * TPW
    sls = [pl.ds(j * 16, 16) for j in range(H // 16)]

    pltpu.sync_copy(ids_hbm.at[pl.ds(base0, TPW)], idw)

    def launch(g, b):
        pltpu.async_copy(w_hbm.at[idw.at[pl.ds(g * CHUNK, CHUNK)]],
                         wbufs[b], sem_w[b])

    def wait_gather(b):
        pltpu.make_async_copy(w_hbm.at[pl.ds(0, CHUNK)], wbufs[b],
                              sem_w[b]).wait()

    def wait_out(b):
        pltpu.make_async_copy(obufs[b], out_hbm.at[pl.ds(0, CHUNK)],
                              sem_o[b]).wait()

    def compute(g, b):
        wb, ob = wbufs[b], obufs[b]
        off = lax.rem(g * CHUNK, L)

        @plsc.parallel_loop(0, CHUNK, unroll=2)
        def row_body(r):
            pos = off + r
            pos = lax.select(pos >= L, pos - L, pos)
            rbase = (r // 16) * 16
            ttv = ttb[pl.ds(g * CHUNK + rbase, 16)]
            tk = _lane_bcast(ttv, r - rbase).astype(_F32)
            ws = [wb[r, sl] for sl in sls]
            ps = [p0res[pos, sl] for sl in sls]
            for j in range(H // 16):
                v = ws[j] + ps[j] + tk * dvals[j]
                ob[r, sls[j]] = jnp.minimum(jnp.maximum(v, -1.0), 1.0)

    def store(g, b):
        pltpu.async_copy(obufs[b], out_hbm.at[pl.ds(base0 + g * CHUNK, CHUNK)],
                         sem_o[b])

    def step(g, b, *, first):
        wait_gather(b)
        if not first:
            wait_out(b)
        compute(g, b)
        store(g, b)

        @pl.when(g + NB < NCHUNK)
        def _():
            launch(g + NB, b)

    for b in range(NB):
        launch(b, b)
    pltpu.sync_copy(tt_hbm.at[pl.ds(base0, TPW)], ttb)
    pltpu.sync_copy(p0_hbm, p0res)
    pltpu.sync_copy(d_hbm, dbuf)
    dvals = [dbuf[sl] for sl in sls]
    for b in range(NB):
        step(b, b, first=True)

    def quad_body(go, carry):
        for b in range(NB):
            step(NB * go + b, b, first=False)
        return carry

    lax.fori_loop(1, NCHUNK // NB, quad_body, 0)

    for b in range(NB):
        wait_out(b)


def kernel(input_ids, attention_mask, token_type_ids, word_embeddings,
           position_embeddings, token_type_embeddings):
    del attention_mask
    ids = input_ids.reshape(TOK).astype(_I32)
    tt = token_type_ids.reshape(TOK).astype(_I32)
    p0 = position_embeddings[:L] + token_type_embeddings[0]
    d = token_type_embeddings[1] - token_type_embeddings[0]
    out = _sc_embed(ids, tt, word_embeddings, p0, d)
    return out.reshape(B, L, H)
